# Initial kernel scaffold; baseline (speedup 1.0000x reference)
import jax
import jax.numpy as jnp
from jax.experimental import pallas as pl


def kernel(data):
    raise NotImplementedError("write your pallas kernel here")


# speedup vs baseline: 106.8469x; 106.8469x over previous
import functools

import jax
import jax.numpy as jnp
from jax import lax
from jax.experimental import pallas as pl
from jax.experimental.pallas import tpu as pltpu
from jax.experimental.pallas import tpu_sc as plsc

_B = 32
_H = 64
_W = 64
_N = _H * _W


def _sort_body(x_ref, ord_ref, key_ref, idx_ref):
    iota = lax.broadcasted_iota(jnp.int32, (_B, _N), 1)
    key_ref[...] = x_ref[...]
    idx_ref[...] = iota

    def stage(s, carry):
        k = jnp.int32(1) << s

        def cex(t, carry2):
            j = k >> (t + 1)
            key = key_ref[...]
            idx = idx_ref[...]
            low = (iota & j) == 0
            asc = (iota & k) == 0
            keep_small = jnp.logical_not(jnp.logical_xor(low, asc))
            sh_neg = jnp.int32(_N) - j
            pkey = jnp.where(low, pltpu.roll(key, sh_neg, 1),
                             pltpu.roll(key, j, 1))
            pidx = jnp.where(low, pltpu.roll(idx, sh_neg, 1),
                             pltpu.roll(idx, j, 1))
            swap = (keep_small & (key > pkey)) | (
                jnp.logical_not(keep_small) & (key < pkey))
            key_ref[...] = jnp.where(swap, pkey, key)
            idx_ref[...] = jnp.where(swap, pidx, idx)
            return carry2

        return lax.fori_loop(0, s, cex, carry)

    lax.fori_loop(1, 13, stage, jnp.int32(0))
    ord_ref[...] = idx_ref[...]


def _argsort_tc(flat):
    return pl.pallas_call(
        _sort_body,
        out_shape=jax.ShapeDtypeStruct((_B, _N), jnp.int32),
        scratch_shapes=[
            pltpu.VMEM((_B, _N), jnp.float32),
            pltpu.VMEM((_B, _N), jnp.int32),
        ],
    )(flat)


def _uf_contrib(flat, order):
    mesh = plsc.VectorSubcoreMesh(core_axis_name="c", subcore_axis_name="s")

    _NP = _N + 16

    @functools.partial(
        pl.kernel,
        mesh=mesh,
        out_type=jax.ShapeDtypeStruct((_B, 16), jnp.float32),
        compiler_params=pltpu.CompilerParams(needs_layout_passes=False),
        scratch_types=[
            pltpu.VMEM((_NP,), jnp.float32),
            pltpu.VMEM((_NP,), jnp.int32),
            pltpu.VMEM((_NP,), jnp.int32),
            pltpu.VMEM((_NP,), jnp.float32),
            pltpu.VMEM((16,), jnp.float32),
        ],
    )
    def uf(vals_hbm, order_hbm, out_hbm, vals_v, order_v, parent_v, len_v,
           row_v):
        b = lax.axis_index("s") * 2 + lax.axis_index("c")
        pltpu.sync_copy(vals_hbm.at[b], vals_v.at[pl.ds(0, _N)])
        pltpu.sync_copy(order_hbm.at[b], order_v.at[pl.ds(0, _N)])

        lane = lax.iota(jnp.int32, 16)
        lane0 = lane == 0
        neg1 = jnp.full((16,), -1, jnp.int32)
        zeros16 = jnp.zeros((16,), jnp.float32)

        def sload(ref, i):
            return ref[pl.ds(i, 16)][0]

        def sstore(ref, i, v):
            plsc.store_scatter(ref, [jnp.full((16,), i, jnp.int32)],
                               jnp.full((16,), v), mask=lane0)

        def init_body(i, carry):
            parent_v[pl.ds(i * 16, 16)] = neg1
            len_v[pl.ds(i * 16, 16)] = zeros16
            return carry

        lax.fori_loop(0, _NP // 16, init_body, jnp.int32(0))

        def find(i):
            return lax.while_loop(lambda r: sload(parent_v, r) != r,
                                  lambda r: sload(parent_v, r), i)

        def step(t, cnt):
            p = sload(order_v, t)
            sstore(parent_v, p, p)
            vp = sload(vals_v, p)
            r0 = p // _W
            c0 = p % _W
            for dr, dc in ((1, 0), (-1, 0), (0, 1), (0, -1)):
                r1 = r0 + dr
                c1 = c0 + dc
                valid = (r1 >= 0) & (r1 < _H) & (c1 >= 0) & (c1 < _W)
                q = jnp.where(valid, r1 * _W + c1, p)
                active = valid & (sload(parent_v, q) != -1)
                rp = find(p)
                sstore(parent_v, p, rp)
                qe = jnp.where(active, q, p)
                rq = find(qe)
                sstore(parent_v, qe, rq)
                merge = active & (rp != rq)
                vrp = sload(vals_v, rp)
                vrq = sload(vals_v, rq)
                rp_elder = (vrp < vrq) | ((vrp == vrq) & (rp < rq))
                young = jnp.where(rp_elder, rq, rp)
                elder = jnp.where(rp_elder, rp, rq)
                length = jnp.where(merge, vp - sload(vals_v, young),
                                   jnp.float32(0.0))
                sstore(parent_v, young,
                       jnp.where(merge, elder, sload(parent_v, young)))
                sstore(len_v, cnt, length)
                cnt = cnt + merge.astype(jnp.int32)
            return cnt

        lax.fori_loop(0, _N, step, jnp.int32(0))

        def topk_body(i, top):
            chunk = len_v[pl.ds(i * 16, 16)]
            cdesc = lax.rev(lax.sort(chunk), (0,))
            return lax.sort(jnp.maximum(top, cdesc))

        top = lax.fori_loop(0, _N // 16, topk_body,
                            jnp.zeros((16,), jnp.float32))

        lane = lax.iota(jnp.int32, 16)
        sq = top * top
        contrib = jnp.where(lane == 15, 1.0 - sq,
                            jnp.where(lane >= 6, sq,
                                      jnp.zeros((16,), jnp.float32)))
        row_v[...] = contrib
        pltpu.sync_copy(row_v, out_hbm.at[b])

    return uf(flat, order)


def kernel(data):
    assert data.shape == (_B, _H, _W), "check the shape!"
    flat = data.reshape(_B, _N)
    order = _argsort_tc(flat)
    contrib = _uf_contrib(flat, order)
    return jnp.sum(contrib)


# speedup vs baseline: 211.0363x; 1.9751x over previous
import functools

import jax
import jax.numpy as jnp
from jax import lax
from jax.experimental import pallas as pl
from jax.experimental.pallas import tpu as pltpu
from jax.experimental.pallas import tpu_sc as plsc

_B = 32
_H = 64
_W = 64
_N = _H * _W


def _sort_body(x_ref, ord_ref, key_ref, idx_ref):
    iota = lax.broadcasted_iota(jnp.int32, (_B, _N), 1)
    key_ref[...] = x_ref[...]
    idx_ref[...] = iota

    def stage(s, carry):
        k = jnp.int32(1) << s

        def cex(t, carry2):
            j = k >> (t + 1)
            key = key_ref[...]
            idx = idx_ref[...]
            low = (iota & j) == 0
            asc = (iota & k) == 0
            keep_small = jnp.logical_not(jnp.logical_xor(low, asc))
            sh_neg = jnp.int32(_N) - j
            pkey = jnp.where(low, pltpu.roll(key, sh_neg, 1),
                             pltpu.roll(key, j, 1))
            pidx = jnp.where(low, pltpu.roll(idx, sh_neg, 1),
                             pltpu.roll(idx, j, 1))
            swap = (keep_small & (key > pkey)) | (
                jnp.logical_not(keep_small) & (key < pkey))
            key_ref[...] = jnp.where(swap, pkey, key)
            idx_ref[...] = jnp.where(swap, pidx, idx)
            return carry2

        return lax.fori_loop(0, s, cex, carry)

    lax.fori_loop(1, 13, stage, jnp.int32(0))
    ord_ref[...] = idx_ref[...]


def _argsort_tc(flat):
    return pl.pallas_call(
        _sort_body,
        out_shape=jax.ShapeDtypeStruct((_B, _N), jnp.int32),
        scratch_shapes=[
            pltpu.VMEM((_B, _N), jnp.float32),
            pltpu.VMEM((_B, _N), jnp.int32),
        ],
    )(flat)


def _uf_contrib(flat, order):
    mesh = plsc.VectorSubcoreMesh(core_axis_name="c", subcore_axis_name="s")

    _NP = _N + 16

    @functools.partial(
        pl.kernel,
        mesh=mesh,
        out_type=jax.ShapeDtypeStruct((_B, 16), jnp.float32),
        compiler_params=pltpu.CompilerParams(needs_layout_passes=False),
        scratch_types=[
            pltpu.VMEM((_NP,), jnp.float32),
            pltpu.VMEM((_NP,), jnp.int32),
            pltpu.VMEM((_NP,), jnp.int32),
            pltpu.VMEM((_NP,), jnp.float32),
            pltpu.VMEM((16,), jnp.float32),
        ],
    )
    def uf(vals_hbm, order_hbm, out_hbm, vals_v, order_v, parent_v, len_v,
           row_v):
        b = lax.axis_index("s") * 2 + lax.axis_index("c")
        pltpu.sync_copy(vals_hbm.at[b], vals_v.at[pl.ds(0, _N)])
        pltpu.sync_copy(order_hbm.at[b], order_v.at[pl.ds(0, _N)])

        lane = lax.iota(jnp.int32, 16)
        lane0 = lane == 0
        neg1 = jnp.full((16,), -1, jnp.int32)
        zeros16 = jnp.zeros((16,), jnp.float32)

        def sload(ref, i):
            return ref[pl.ds(i, 16)][0]

        def sstore(ref, i, v):
            plsc.store_scatter(ref, [jnp.full((16,), i, jnp.int32)],
                               jnp.full((16,), v), mask=lane0)

        def init_body(i, carry):
            parent_v[pl.ds(i * 16, 16)] = neg1
            len_v[pl.ds(i * 16, 16)] = zeros16
            return carry

        lax.fori_loop(0, _NP // 16, init_body, jnp.int32(0))

        def step(t, cnt):
            p = sload(order_v, t)
            sstore(parent_v, p, p)
            vp = sload(vals_v, p)
            r0 = p // _W
            c0 = p % _W

            def visit(carry, q, active):
                def active_fn(carry):
                    rp, cnt = carry
                    rq = lax.while_loop(
                        lambda r: sload(parent_v, r) != r,
                        lambda r: sload(parent_v, r), sload(parent_v, q))
                    sstore(parent_v, q, rq)
                    merge = rq != rp
                    vrp = sload(vals_v, rp)
                    vrq = sload(vals_v, rq)
                    rp_elder = (vrp < vrq) | ((vrp == vrq) & (rp < rq))
                    young = jnp.where(rp_elder, rq, rp)
                    elder = jnp.where(rp_elder, rp, rq)
                    length = jnp.where(merge, vp - jnp.maximum(vrp, vrq),
                                       jnp.float32(0.0))
                    sstore(parent_v, young, elder)
                    sstore(len_v, cnt, length)
                    return (jnp.where(merge, elder, rp),
                            cnt + merge.astype(jnp.int32))

                return lax.cond(active, active_fn, lambda c: c, carry)

            carry = (p, cnt)
            for dr, dc in ((1, 0), (-1, 0), (0, 1), (0, -1)):
                r1 = r0 + dr
                c1 = c0 + dc
                valid = (r1 >= 0) & (r1 < _H) & (c1 >= 0) & (c1 < _W)
                q = jnp.where(valid, r1 * _W + c1, p)
                active = valid & (sload(parent_v, q) != -1)
                carry = visit(carry, q, active)
            return carry[1]

        lax.fori_loop(0, _N, step, jnp.int32(0))

        def topk_body(i, top):
            chunk = len_v[pl.ds(i * 16, 16)]
            cdesc = lax.rev(lax.sort(chunk), (0,))
            return lax.sort(jnp.maximum(top, cdesc))

        top = lax.fori_loop(0, _N // 16, topk_body,
                            jnp.zeros((16,), jnp.float32))

        lane = lax.iota(jnp.int32, 16)
        sq = top * top
        contrib = jnp.where(lane == 15, 1.0 - sq,
                            jnp.where(lane >= 6, sq,
                                      jnp.zeros((16,), jnp.float32)))
        row_v[...] = contrib
        pltpu.sync_copy(row_v, out_hbm.at[b])

    return uf(flat, order)


def kernel(data):
    assert data.shape == (_B, _H, _W), "check the shape!"
    flat = data.reshape(_B, _N)
    order = _argsort_tc(flat)
    contrib = _uf_contrib(flat, order)
    return jnp.sum(contrib)
